# Initial kernel scaffold; baseline (speedup 1.0000x reference)
import jax
import jax.numpy as jnp
from jax.experimental import pallas as pl


def kernel(x, pos, edge_index, Wq, Wsim, Wk1, Wk2, Wv1, Wv2):
    raise NotImplementedError("write your pallas kernel here")


# speedup vs baseline: 1.0512x; 1.0512x over previous
import functools
import numpy as np
import jax
import jax.numpy as jnp
from jax.experimental import pallas as pl
from jax.experimental.pallas import tpu as pltpu

MUL_ = 8
KDIM_ = 8
VDIM_ = 8
NB_ = 32
RC_ = 2.5
EBLK = 1280


def _dense_body(esrc_ref, edst_ref, wq_ref, wsim_ref, wk1_ref, wk2_ref,
                wv1_ref, wv2_ref, sim_ref, vt_ref):
    esrc = esrc_ref[...]
    edst = edst_ref[...]
    x0 = esrc[:, 0:8]
    x1x = esrc[:, 8:16]
    x1y = esrc[:, 16:24]
    x1z = esrc[:, 24:32]
    pos_s = esrc[:, 32:35]
    pos_d = edst[:, 0:3]
    xd8 = edst[:, 3:11]

    vec = pos_s - pos_d
    r2 = jnp.sum(vec * vec, axis=1, keepdims=True)
    r = jnp.sqrt(jnp.maximum(r2, 1e-12))
    u = vec / r

    roots = (jax.lax.broadcasted_iota(jnp.int32, (1, NB_), 1).astype(jnp.float32) + 1.0) * np.pi
    r_safe = jnp.where(r > 1e-6, r, 1.0)
    mask = jnp.logical_and(r > 1e-6, r < RC_).astype(jnp.float32)
    emb = (np.sqrt(2.0 / RC_) * np.sqrt(float(NB_))) * jnp.sin(roots * (r / RC_)) / r_safe * mask

    h_k = jax.nn.silu(jnp.dot(emb, wk1_ref[...], preferred_element_type=jnp.float32) / np.sqrt(float(NB_)))
    wk = jnp.dot(h_k, wk2_ref[...], preferred_element_type=jnp.float32) / 4.0
    h_v = jax.nn.silu(jnp.dot(emb, wv1_ref[...], preferred_element_type=jnp.float32) / np.sqrt(float(NB_)))
    wv = jnp.dot(h_v, wv2_ref[...], preferred_element_type=jnp.float32) / 4.0

    ux = u[:, 0:1]
    uy = u[:, 1:2]
    uz = u[:, 2:3]
    s = x1x * ux + x1y * uy + x1z * uz

    k = jnp.zeros_like(x0)
    v = jnp.zeros_like(x0)
    for uu in range(MUL_):
        xc = x0[:, uu:uu + 1]
        sc = s[:, uu:uu + 1]
        k = k + xc * wk[:, uu * 8:(uu + 1) * 8] + sc * wk[:, 64 + uu * 8:64 + (uu + 1) * 8]
        v = v + xc * wv[:, uu * 8:(uu + 1) * 8] + sc * wv[:, 64 + uu * 8:64 + (uu + 1) * 8]
    k = k / 4.0
    v = v / 4.0

    wqs = jnp.dot(wq_ref[...], wsim_ref[...], preferred_element_type=jnp.float32)
    qs = jnp.dot(xd8, wqs, preferred_element_type=jnp.float32)
    sim = jnp.sum(qs * k, axis=1) / (np.sqrt(8.0) * 8.0)

    sim_ref[...] = sim[None, :]
    vt_ref[...] = v.T


def _dense_call(esrc, edst, Wq, Wsim, Wk1, Wk2, Wv1, Wv2):
    E = esrc.shape[0]
    grid = E // EBLK
    full = lambda shp: pl.BlockSpec(shp, lambda i: (0,) * len(shp))
    return pl.pallas_call(
        _dense_body,
        grid=(grid,),
        in_specs=[
            pl.BlockSpec((EBLK, 48), lambda i: (i, 0)),
            pl.BlockSpec((EBLK, 16), lambda i: (i, 0)),
            full((8, 8)), full((8, 8)),
            full((NB_, 16)), full((16, 128)),
            full((NB_, 16)), full((16, 128)),
        ],
        out_specs=[
            pl.BlockSpec((1, EBLK), lambda i: (0, i)),
            pl.BlockSpec((8, EBLK), lambda i: (0, i)),
        ],
        out_shape=[
            jax.ShapeDtypeStruct((1, E), jnp.float32),
            jax.ShapeDtypeStruct((8, E), jnp.float32),
        ],
    )(esrc, edst, Wq, Wsim, Wk1, Wk2, Wv1, Wv2)


def kernel(x, pos, edge_index, Wq, Wsim, Wk1, Wk2, Wv1, Wv2):
    n = x.shape[0]
    src = edge_index[0]
    dst = edge_index[1]

    x0 = x[:, :MUL_]
    x1 = x[:, MUL_:].reshape(n, MUL_, 3)
    src_table = jnp.concatenate(
        [x0, x1[:, :, 0], x1[:, :, 1], x1[:, :, 2], pos,
         jnp.zeros((n, 13), jnp.float32)], axis=1)
    dst_table = jnp.concatenate(
        [pos, x[:, :MUL_], jnp.zeros((n, 5), jnp.float32)], axis=1)

    esrc = jnp.take(src_table, src, axis=0)
    edst = jnp.take(dst_table, dst, axis=0)

    sim, vt = _dense_call(esrc, edst, Wq, Wsim, Wk1, Wk2, Wv1, Wv2)
    sim = sim[0]
    v = vt.T

    m = jax.ops.segment_max(sim, dst, num_segments=n)
    t = jnp.exp((sim - m[dst]) * 0.5)
    denom = jax.ops.segment_sum(t * t, dst, num_segments=n)
    numer = jax.ops.segment_sum(t[:, None] * v, dst, num_segments=n)
    return numer * jnp.where(denom > 0, jax.lax.rsqrt(denom), 0.0)[:, None]


# speedup vs baseline: 1.2381x; 1.1778x over previous
import functools
import numpy as np
import jax
import jax.numpy as jnp
from jax import lax
from jax.experimental import pallas as pl
from jax.experimental.pallas import tpu as pltpu
from jax.experimental.pallas import tpu_sc as plsc

MUL_ = 8
KDIM_ = 8
VDIM_ = 8
NB_ = 32
RC_ = 2.5
EBLK = 1280
NEDGE = 160000
NWORK = 32
EPW = NEDGE // NWORK
GCH = 1000
NCH = EPW // GCH


def _sc_gather(src_tab, dst_tab, src_idx, dst_idx, esrc_out, edst_out,
               sidx_v, srow_v, didx_v, drow_v, sem1, sem2):
    nc = 2
    wid = lax.axis_index("s") * nc + lax.axis_index("c")

    def body(g, carry):
        base = wid * EPW + g * GCH
        pltpu.sync_copy(src_idx.at[pl.ds(base, GCH)], sidx_v)
        pltpu.sync_copy(dst_idx.at[pl.ds(base, GCH)], didx_v)
        cp1 = pltpu.async_copy(src_tab.at[sidx_v], srow_v, sem1)
        cp2 = pltpu.async_copy(dst_tab.at[didx_v], drow_v, sem2)
        cp1.wait()
        cp2.wait()
        pltpu.sync_copy(srow_v, esrc_out.at[pl.ds(base, GCH)])
        pltpu.sync_copy(drow_v, edst_out.at[pl.ds(base, GCH)])
        return carry

    lax.fori_loop(0, NCH, body, 0)


def _gather_call(src_table, dst_table, src, dst):
    n = src_table.shape[0]
    mesh = plsc.VectorSubcoreMesh(core_axis_name="c", subcore_axis_name="s")
    k = functools.partial(
        pl.kernel, mesh=mesh,
        compiler_params=pltpu.CompilerParams(use_tc_tiling_on_sc=False),
        out_type=[
            jax.ShapeDtypeStruct((NEDGE, 48), jnp.float32),
            jax.ShapeDtypeStruct((NEDGE, 16), jnp.float32),
        ],
        scratch_types=[
            pltpu.VMEM((GCH,), jnp.int32),
            pltpu.VMEM((GCH, 48), jnp.float32),
            pltpu.VMEM((GCH,), jnp.int32),
            pltpu.VMEM((GCH, 16), jnp.float32),
            pltpu.SemaphoreType.DMA,
            pltpu.SemaphoreType.DMA,
        ],
    )(_sc_gather)
    return k(src_table, dst_table, src, dst)


def _dense_body(esrc_ref, edst_ref, wq_ref, wsim_ref, wk1_ref, wk2_ref,
                wv1_ref, wv2_ref, sim_ref, vt_ref):
    esrc = esrc_ref[...]
    edst = edst_ref[...]
    x0 = esrc[:, 0:8]
    x1x = esrc[:, 8:16]
    x1y = esrc[:, 16:24]
    x1z = esrc[:, 24:32]
    pos_s = esrc[:, 32:35]
    pos_d = edst[:, 0:3]
    xd8 = edst[:, 3:11]

    vec = pos_s - pos_d
    r2 = jnp.sum(vec * vec, axis=1, keepdims=True)
    r = jnp.sqrt(jnp.maximum(r2, 1e-12))
    u = vec / r

    roots = (jax.lax.broadcasted_iota(jnp.int32, (1, NB_), 1).astype(jnp.float32) + 1.0) * np.pi
    r_safe = jnp.where(r > 1e-6, r, 1.0)
    mask = jnp.logical_and(r > 1e-6, r < RC_).astype(jnp.float32)
    emb = (np.sqrt(2.0 / RC_) * np.sqrt(float(NB_))) * jnp.sin(roots * (r / RC_)) / r_safe * mask

    h_k = jax.nn.silu(jnp.dot(emb, wk1_ref[...], preferred_element_type=jnp.float32) / np.sqrt(float(NB_)))
    wk = jnp.dot(h_k, wk2_ref[...], preferred_element_type=jnp.float32) / 4.0
    h_v = jax.nn.silu(jnp.dot(emb, wv1_ref[...], preferred_element_type=jnp.float32) / np.sqrt(float(NB_)))
    wv = jnp.dot(h_v, wv2_ref[...], preferred_element_type=jnp.float32) / 4.0

    ux = u[:, 0:1]
    uy = u[:, 1:2]
    uz = u[:, 2:3]
    s = x1x * ux + x1y * uy + x1z * uz

    k = jnp.zeros_like(x0)
    v = jnp.zeros_like(x0)
    for uu in range(MUL_):
        xc = x0[:, uu:uu + 1]
        sc = s[:, uu:uu + 1]
        k = k + xc * wk[:, uu * 8:(uu + 1) * 8] + sc * wk[:, 64 + uu * 8:64 + (uu + 1) * 8]
        v = v + xc * wv[:, uu * 8:(uu + 1) * 8] + sc * wv[:, 64 + uu * 8:64 + (uu + 1) * 8]
    k = k / 4.0
    v = v / 4.0

    wqs = jnp.dot(wq_ref[...], wsim_ref[...], preferred_element_type=jnp.float32)
    qs = jnp.dot(xd8, wqs, preferred_element_type=jnp.float32)
    sim = jnp.sum(qs * k, axis=1) / (np.sqrt(8.0) * 8.0)

    sim_ref[...] = sim[None, :]
    vt_ref[...] = v.T


def _dense_call(esrc, edst, Wq, Wsim, Wk1, Wk2, Wv1, Wv2):
    E = esrc.shape[0]
    grid = E // EBLK
    full = lambda shp: pl.BlockSpec(shp, lambda i: (0,) * len(shp))
    return pl.pallas_call(
        _dense_body,
        grid=(grid,),
        in_specs=[
            pl.BlockSpec((EBLK, 48), lambda i: (i, 0)),
            pl.BlockSpec((EBLK, 16), lambda i: (i, 0)),
            full((8, 8)), full((8, 8)),
            full((NB_, 16)), full((16, 128)),
            full((NB_, 16)), full((16, 128)),
        ],
        out_specs=[
            pl.BlockSpec((1, EBLK), lambda i: (0, i)),
            pl.BlockSpec((8, EBLK), lambda i: (0, i)),
        ],
        out_shape=[
            jax.ShapeDtypeStruct((1, E), jnp.float32),
            jax.ShapeDtypeStruct((8, E), jnp.float32),
        ],
    )(esrc, edst, Wq, Wsim, Wk1, Wk2, Wv1, Wv2)


def kernel(x, pos, edge_index, Wq, Wsim, Wk1, Wk2, Wv1, Wv2):
    n = x.shape[0]
    src = edge_index[0]
    dst = edge_index[1]

    x0 = x[:, :MUL_]
    x1 = x[:, MUL_:].reshape(n, MUL_, 3)
    src_table = jnp.concatenate(
        [x0, x1[:, :, 0], x1[:, :, 1], x1[:, :, 2], pos,
         jnp.zeros((n, 13), jnp.float32)], axis=1)
    dst_table = jnp.concatenate(
        [pos, x[:, :MUL_], jnp.zeros((n, 5), jnp.float32)], axis=1)

    esrc, edst = _gather_call(src_table, dst_table, src, dst)

    sim, vt = _dense_call(esrc, edst, Wq, Wsim, Wk1, Wk2, Wv1, Wv2)
    sim = sim[0]
    v = vt.T

    m = jax.ops.segment_max(sim, dst, num_segments=n)
    t = jnp.exp((sim - m[dst]) * 0.5)
    denom = jax.ops.segment_sum(t * t, dst, num_segments=n)
    numer = jax.ops.segment_sum(t[:, None] * v, dst, num_segments=n)
    return numer * jnp.where(denom > 0, jax.lax.rsqrt(denom), 0.0)[:, None]


# speedup vs baseline: 1.9455x; 1.5713x over previous
import functools
import numpy as np
import jax
import jax.numpy as jnp
from jax import lax
from jax.experimental import pallas as pl
from jax.experimental.pallas import tpu as pltpu
from jax.experimental.pallas import tpu_sc as plsc

MUL_ = 8
KDIM_ = 8
VDIM_ = 8
NB_ = 32
RC_ = 2.5
EBLK = 1280
NEDGE = 160000
NEPAD = 161280
NWORK = 32
EPW = NEPAD // NWORK
GCH = 1008
NCH = EPW // GCH
NNPAD = 10240
NPT = NNPAD // 16
SCH = 80
NSCH = EPW // SCH


def _sc_gather(src_tab, dst_tab, src_idx, dst_idx, esrc_out, edst_out,
               sidx_v, srow_v, didx_v, drow_v, sem1, sem2):
    nc = 2
    wid = lax.axis_index("s") * nc + lax.axis_index("c")

    def body(g, carry):
        base = wid * EPW + g * GCH
        pltpu.sync_copy(src_idx.at[pl.ds(base, GCH)], sidx_v)
        pltpu.sync_copy(dst_idx.at[pl.ds(base, GCH)], didx_v)
        cp1 = pltpu.async_copy(src_tab.at[sidx_v], srow_v, sem1)
        cp2 = pltpu.async_copy(dst_tab.at[didx_v], drow_v, sem2)
        cp1.wait()
        cp2.wait()
        pltpu.sync_copy(srow_v, esrc_out.at[pl.ds(base, GCH)])
        pltpu.sync_copy(drow_v, edst_out.at[pl.ds(base, GCH)])
        return carry

    lax.fori_loop(0, NCH, body, 0)


def _gather_call(src_table, dst_table, src, dst):
    n = src_table.shape[0]
    mesh = plsc.VectorSubcoreMesh(core_axis_name="c", subcore_axis_name="s")
    k = functools.partial(
        pl.kernel, mesh=mesh,
        compiler_params=pltpu.CompilerParams(use_tc_tiling_on_sc=False, needs_layout_passes=False),
        out_type=[
            jax.ShapeDtypeStruct((NEPAD, 48), jnp.float32),
            jax.ShapeDtypeStruct((NEPAD, 16), jnp.float32),
        ],
        scratch_types=[
            pltpu.VMEM((GCH,), jnp.int32),
            pltpu.VMEM((GCH, 48), jnp.float32),
            pltpu.VMEM((GCH,), jnp.int32),
            pltpu.VMEM((GCH, 16), jnp.float32),
            pltpu.SemaphoreType.DMA,
            pltpu.SemaphoreType.DMA,
        ],
    )(_sc_gather)
    return k(src_table, dst_table, src, dst)


def _vtake(x, idx):
    dnums = lax.GatherDimensionNumbers(
        offset_dims=(), collapsed_slice_dims=(0,), start_index_map=(0,))
    return lax.gather(x, idx[:, None], dnums, slice_sizes=(1,),
                      mode=lax.GatherScatterMode.PROMISE_IN_BOUNDS)


def _runmax_scatter_prep(d, sv):
    ks, vs = plsc.sort_key_val(d, sv)
    lane = lax.iota(jnp.int32, 16)
    for sft in (1, 2, 4, 8):
        j2 = jnp.maximum(lane - sft, 0)
        kv = _vtake(ks, j2)
        xv = _vtake(vs, j2)
        vs = jnp.where(jnp.logical_and(kv == ks, lane >= sft),
                       jnp.maximum(vs, xv), vs)
    nxt = _vtake(ks, jnp.minimum(lane + 1, 15))
    is_end = jnp.logical_or(lane == 15, nxt != ks)
    return ks, vs, is_end


def _sc_segmax(dst_idx, sim_in, m_out, mloc, dstc, simc, shared, sem):
    nc = 2
    cid = lax.axis_index("c")
    sid = lax.axis_index("s")
    wid = sid * nc + cid
    neg = jnp.full((16,), -1e30, jnp.float32)

    def initb(i, c):
        mloc[pl.ds(i * 16, 16)] = neg
        return c
    lax.fori_loop(0, NNPAD // 16, initb, 0)

    def chunk(g, c):
        base = wid * EPW + g * GCH
        pltpu.sync_copy(dst_idx.at[pl.ds(base, GCH)], dstc)
        pltpu.sync_copy(sim_in.at[pl.ds(base, GCH)], simc)

        def grp(j, c2):
            d = dstc[pl.ds(j * 16, 16)]
            sv = simc[pl.ds(j * 16, 16)]
            ks, vs, is_end = _runmax_scatter_prep(d, sv)
            cur = plsc.load_gather(mloc, [ks])
            plsc.store_scatter(mloc, [ks], jnp.maximum(cur, vs), mask=is_end)
            return c2
        lax.fori_loop(0, GCH // 16, grp, 0)
        return c
    lax.fori_loop(0, NCH, chunk, 0)

    if True:
        pltpu.sync_copy(mloc, shared.at[sid])
        plsc.subcore_barrier()
        off = sid * NPT
        def red(t, c):
            pltpu.sync_copy(shared.at[t, pl.ds(off, NPT)], simc.at[pl.ds(0, NPT)])
            def mx(i, c2):
                sl = pl.ds(off + i * 16, 16)
                mloc[sl] = jnp.maximum(mloc[sl], simc[pl.ds(i * 16, 16)])
                return c2
            lax.fori_loop(0, NPT // 16, mx, 0)
            return c
        lax.fori_loop(0, 16, red, 0)
        pltpu.sync_copy(mloc.at[pl.ds(off, NPT)], m_out.at[cid, pl.ds(off, NPT)])


def _segmax_call(dst, sim):
    mesh = plsc.VectorSubcoreMesh(core_axis_name="c", subcore_axis_name="s")
    k = functools.partial(
        pl.kernel, mesh=mesh,
        compiler_params=pltpu.CompilerParams(use_tc_tiling_on_sc=False, needs_layout_passes=False),
        out_type=jax.ShapeDtypeStruct((2, NNPAD), jnp.float32),
        scratch_types=[
            pltpu.VMEM((NNPAD,), jnp.float32),
            pltpu.VMEM((GCH,), jnp.int32),
            pltpu.VMEM((GCH,), jnp.float32),
            pltpu.VMEM_SHARED((16, NNPAD), jnp.float32),
            pltpu.SemaphoreType.DMA,
        ],
    )(_sc_segmax)
    return k(dst, sim)


def _sc_scatter_add(dst_idx, sim_in, vt_in, m2_in, zrows, part_out,
                    mfull, tmpm, dstc, simc, vbuf, block, outbuf, acc, sem):
    nc = 2
    cid = lax.axis_index("c")
    sid = lax.axis_index("s")
    wid = sid * nc + cid
    lane = lax.iota(jnp.int32, 16)

    pltpu.sync_copy(m2_in.at[0], mfull)
    pltpu.sync_copy(m2_in.at[1], tmpm)

    def mrg(i, c):
        sl = pl.ds(i * 16, 16)
        mfull[sl] = jnp.maximum(mfull[sl], tmpm[sl])
        return c
    lax.fori_loop(0, NNPAD // 16, mrg, 0)

    pltpu.sync_copy(zrows.at[pl.ds(0, SCH)], block)

    if True:
        pltpu.sync_copy(zrows, acc.at[pl.ds(sid * NPT, NPT)])
        plsc.subcore_barrier()

        def chunk(g, c):
            base = wid * EPW + g * SCH
            pltpu.sync_copy(dst_idx.at[pl.ds(base, SCH)], dstc)
            pltpu.sync_copy(sim_in.at[pl.ds(base, SCH)], simc)
            for comp in range(8):
                pltpu.sync_copy(vt_in.at[comp, pl.ds(base, SCH)],
                                vbuf.at[comp])

            def grp(j, c2):
                off = j * 16
                d = dstc[pl.ds(off, 16)]
                sv = simc[pl.ds(off, 16)]
                mv = plsc.load_gather(mfull, [d])
                t = jnp.exp((sv - mv) * 0.5)
                rows = off + lane
                for comp in range(8):
                    tv = t * vbuf[comp, pl.ds(off, 16)]
                    cc = jnp.full((16,), comp, jnp.int32)
                    plsc.store_scatter(block, [rows, cc], tv)
                plsc.store_scatter(block, [rows, jnp.full((16,), 8, jnp.int32)],
                                   t * t)
                return c2
            lax.fori_loop(0, SCH // 16, grp, 0)
            pltpu.sync_copy(block, acc.at[dstc], add=True)
            return c
        lax.fori_loop(0, NSCH, chunk, 0)

        plsc.subcore_barrier()
        off = sid * NPT
        pltpu.sync_copy(acc.at[pl.ds(off, NPT)], outbuf)
        pltpu.sync_copy(outbuf, part_out.at[cid, pl.ds(off, NPT)])


def _scatter_add_call(dst, sim, vt, m2):
    mesh = plsc.VectorSubcoreMesh(core_axis_name="c", subcore_axis_name="s")
    zrows = jnp.zeros((NPT, 16), jnp.float32)
    k = functools.partial(
        pl.kernel, mesh=mesh,
        compiler_params=pltpu.CompilerParams(use_tc_tiling_on_sc=False, needs_layout_passes=False),
        out_type=jax.ShapeDtypeStruct((2, NNPAD, 16), jnp.float32),
        scratch_types=[
            pltpu.VMEM((NNPAD,), jnp.float32),
            pltpu.VMEM((NNPAD,), jnp.float32),
            pltpu.VMEM((SCH,), jnp.int32),
            pltpu.VMEM((SCH,), jnp.float32),
            pltpu.VMEM((8, SCH), jnp.float32),
            pltpu.VMEM((SCH, 16), jnp.float32),
            pltpu.VMEM((NPT, 16), jnp.float32),
            pltpu.VMEM_SHARED((NNPAD, 16), jnp.float32),
            pltpu.SemaphoreType.DMA,
        ],
    )(_sc_scatter_add)
    return k(dst, sim, vt, m2, zrows)


def _final_body(part_ref, out_ref):
    p0 = part_ref[0]
    p1 = part_ref[1]
    numer = p0[:10000, :8] + p1[:10000, :8]
    denom = p0[:10000, 8:9] + p1[:10000, 8:9]
    coef = jnp.where(denom > 0, jax.lax.rsqrt(jnp.where(denom > 0, denom, 1.0)), 0.0)
    out_ref[...] = numer * coef


def _final_call(part):
    return pl.pallas_call(
        _final_body,
        out_shape=jax.ShapeDtypeStruct((10000, 8), jnp.float32),
    )(part)


def _dense_body(esrc_ref, edst_ref, wq_ref, wsim_ref, wk1_ref, wk2_ref,
                wv1_ref, wv2_ref, sim_ref, vt_ref):
    esrc = esrc_ref[...]
    edst = edst_ref[...]
    x0 = esrc[:, 0:8]
    x1x = esrc[:, 8:16]
    x1y = esrc[:, 16:24]
    x1z = esrc[:, 24:32]
    pos_s = esrc[:, 32:35]
    pos_d = edst[:, 0:3]
    xd8 = edst[:, 3:11]

    vec = pos_s - pos_d
    r2 = jnp.sum(vec * vec, axis=1, keepdims=True)
    r = jnp.sqrt(jnp.maximum(r2, 1e-12))
    u = vec / r

    roots = (jax.lax.broadcasted_iota(jnp.int32, (1, NB_), 1).astype(jnp.float32) + 1.0) * np.pi
    r_safe = jnp.where(r > 1e-6, r, 1.0)
    mask = jnp.logical_and(r > 1e-6, r < RC_).astype(jnp.float32)
    emb = (np.sqrt(2.0 / RC_) * np.sqrt(float(NB_))) * jnp.sin(roots * (r / RC_)) / r_safe * mask

    h_k = jax.nn.silu(jnp.dot(emb, wk1_ref[...], preferred_element_type=jnp.float32) / np.sqrt(float(NB_)))
    wk = jnp.dot(h_k, wk2_ref[...], preferred_element_type=jnp.float32) / 4.0
    h_v = jax.nn.silu(jnp.dot(emb, wv1_ref[...], preferred_element_type=jnp.float32) / np.sqrt(float(NB_)))
    wv = jnp.dot(h_v, wv2_ref[...], preferred_element_type=jnp.float32) / 4.0

    ux = u[:, 0:1]
    uy = u[:, 1:2]
    uz = u[:, 2:3]
    s = x1x * ux + x1y * uy + x1z * uz

    k = jnp.zeros_like(x0)
    v = jnp.zeros_like(x0)
    for uu in range(MUL_):
        xc = x0[:, uu:uu + 1]
        sc = s[:, uu:uu + 1]
        k = k + xc * wk[:, uu * 8:(uu + 1) * 8] + sc * wk[:, 64 + uu * 8:64 + (uu + 1) * 8]
        v = v + xc * wv[:, uu * 8:(uu + 1) * 8] + sc * wv[:, 64 + uu * 8:64 + (uu + 1) * 8]
    k = k / 4.0
    v = v / 4.0

    qd = jnp.dot(xd8, wq_ref[...], preferred_element_type=jnp.float32) / np.sqrt(8.0)
    qw = jnp.dot(qd, wsim_ref[...], preferred_element_type=jnp.float32)
    sim = jnp.sum(qw * k, axis=1) / 8.0

    gid = pl.program_id(0) * EBLK + jax.lax.broadcasted_iota(jnp.int32, (EBLK,), 0)
    sim = jnp.where(gid < NEDGE, sim, -1e30)

    sim_ref[...] = sim[None, :]
    vt_ref[...] = v.T


def _dense_call(esrc, edst, Wq, Wsim, Wk1, Wk2, Wv1, Wv2):
    E = esrc.shape[0]
    grid = E // EBLK
    full = lambda shp: pl.BlockSpec(shp, lambda i: (0,) * len(shp))
    return pl.pallas_call(
        _dense_body,
        grid=(grid,),
        in_specs=[
            pl.BlockSpec((EBLK, 48), lambda i: (i, 0)),
            pl.BlockSpec((EBLK, 16), lambda i: (i, 0)),
            full((8, 8)), full((8, 8)),
            full((NB_, 16)), full((16, 128)),
            full((NB_, 16)), full((16, 128)),
        ],
        out_specs=[
            pl.BlockSpec((1, EBLK), lambda i: (0, i)),
            pl.BlockSpec((8, EBLK), lambda i: (0, i)),
        ],
        out_shape=[
            jax.ShapeDtypeStruct((1, E), jnp.float32),
            jax.ShapeDtypeStruct((8, E), jnp.float32),
        ],
        compiler_params=pltpu.CompilerParams(
            dimension_semantics=("arbitrary",)),
    )(esrc, edst, Wq, Wsim, Wk1, Wk2, Wv1, Wv2)


def kernel(x, pos, edge_index, Wq, Wsim, Wk1, Wk2, Wv1, Wv2):
    n = x.shape[0]
    pad = jnp.zeros((NEPAD - NEDGE,), jnp.int32)
    src = jnp.concatenate([edge_index[0], pad])
    dst = jnp.concatenate([edge_index[1], pad])

    x0 = x[:, :MUL_]
    x1 = x[:, MUL_:].reshape(n, MUL_, 3)
    src_table = jnp.concatenate(
        [x0, x1[:, :, 0], x1[:, :, 1], x1[:, :, 2], pos,
         jnp.zeros((n, 13), jnp.float32)], axis=1)
    dst_table = jnp.concatenate(
        [pos, x[:, :MUL_], jnp.zeros((n, 5), jnp.float32)], axis=1)

    esrc, edst = _gather_call(src_table, dst_table, src, dst)
    sim2, vt = _dense_call(esrc, edst, Wq, Wsim, Wk1, Wk2, Wv1, Wv2)
    sim = sim2.reshape(NEPAD)
    m2 = _segmax_call(dst, sim)
    part = _scatter_add_call(dst, sim, vt, m2)
    return _final_call(part)
